# baseline (speedup 1.0000x reference)
import jax
import jax.numpy as jnp
from jax import lax
from jax.experimental import pallas as pl
from jax.experimental.pallas import tpu as pltpu
from jax.experimental.pallas import tpu_sc as plsc

_NUM_BUCKETS = 1000000
_EMBED_DIM = 64
_BATCH = 16384

_info = plsc.get_sparse_core_info()
_NC, _NS, _L = _info.num_cores, _info.num_subcores, _info.num_lanes
_NW = _NC * _NS
_B_PER_W = _BATCH // _NW
_CH = 128
_N_CH = _B_PER_W // _CH


def _embed_body(x_hbm, table_hbm, out_hbm, idx_v, rows_v, sem):
    wid = lax.axis_index("s") * _NC + lax.axis_index("c")
    base = wid * _B_PER_W

    for j in range(_N_CH):
        pltpu.sync_copy(x_hbm.at[pl.ds(base + j * _CH, _CH)], idx_v.at[j])

    for j in range(_N_CH):
        for i in range(_CH // _L):
            sl = pl.ds(i * _L, _L)
            idx_v[j, sl] = lax.rem(idx_v[j, sl], _NUM_BUCKETS)

    copies = [
        pltpu.async_copy(table_hbm.at[idx_v.at[j]], rows_v.at[j], sem)
        for j in range(_N_CH)
    ]
    for c in copies:
        c.wait()

    for j in range(_N_CH):
        pltpu.sync_copy(rows_v.at[j], out_hbm.at[pl.ds(base + j * _CH, _CH)])


@jax.jit
def kernel(x, table):
    fn = pl.kernel(
        _embed_body,
        out_type=jax.ShapeDtypeStruct((_BATCH, _EMBED_DIM), jnp.float32),
        mesh=plsc.VectorSubcoreMesh(core_axis_name="c", subcore_axis_name="s"),
        scratch_types=[
            pltpu.VMEM((_N_CH, _CH), jnp.int32),
            pltpu.VMEM((_N_CH, _CH, _EMBED_DIM), jnp.float32),
            pltpu.SemaphoreType.DMA,
        ],
        compiler_params=pltpu.CompilerParams(use_tc_tiling_on_sc=False),
    )
    return fn(x, table)


# speedup vs baseline: 1.0357x; 1.0357x over previous
import jax
import jax.numpy as jnp
from jax import lax
from jax.experimental import pallas as pl
from jax.experimental.pallas import tpu as pltpu
from jax.experimental.pallas import tpu_sc as plsc

_NUM_BUCKETS = 1000000
_EMBED_DIM = 64
_BATCH = 16384

_info = plsc.get_sparse_core_info()
_NC, _NS, _L = _info.num_cores, _info.num_subcores, _info.num_lanes
_NW = _NC * _NS
_B_PER_W = _BATCH // _NW


def _embed_body(x_hbm, table_hbm, out_hbm, idx_v, sem):
    wid = lax.axis_index("s") * _NC + lax.axis_index("c")
    base = wid * _B_PER_W

    pltpu.sync_copy(x_hbm.at[pl.ds(base, _B_PER_W)], idx_v)
    for i in range(_B_PER_W // _L):
        sl = pl.ds(i * _L, _L)
        idx_v[sl] = lax.rem(idx_v[sl], _NUM_BUCKETS)

    def group(g, carry):
        chunk = idx_v[pl.ds(g * _L, _L)]
        gbase = base + g * _L
        for lane in range(_L):
            r = chunk[lane]
            pltpu.async_copy(
                table_hbm.at[pl.ds(r, 1)], out_hbm.at[pl.ds(gbase + lane, 1)], sem
            )
        return carry

    lax.fori_loop(0, _B_PER_W // _L, group, 0)

    def drain(i, carry):
        pltpu.make_async_copy(
            table_hbm.at[pl.ds(0, 1)], out_hbm.at[pl.ds(base, 1)], sem
        ).wait()
        return carry

    lax.fori_loop(0, _B_PER_W, drain, 0)


@jax.jit
def kernel(x, table):
    fn = pl.kernel(
        _embed_body,
        out_type=jax.ShapeDtypeStruct((_BATCH, _EMBED_DIM), jnp.float32),
        mesh=plsc.VectorSubcoreMesh(core_axis_name="c", subcore_axis_name="s"),
        scratch_types=[
            pltpu.VMEM((_B_PER_W,), jnp.int32),
            pltpu.SemaphoreType.DMA,
        ],
    )
    return fn(x, table)


# speedup vs baseline: 3.0834x; 2.9770x over previous
import jax
import jax.numpy as jnp
from jax import lax
from jax.experimental import pallas as pl
from jax.experimental.pallas import tpu as pltpu
from jax.experimental.pallas import tpu_sc as plsc

_NUM_BUCKETS = 1000000
_EMBED_DIM = 64
_BATCH = 16384

_info = plsc.get_sparse_core_info()
_NC, _NS, _L = _info.num_cores, _info.num_subcores, _info.num_lanes
_NW = _NC * _NS
_B_PER_W = _BATCH // _NW
_QS = 4
_NQ = _B_PER_W // _QS
_IDX_PAD = _B_PER_W + _L


def _embed_body(x_hbm, tableT_hbm, outT_hbm, idx_v, blocks, stage, sem):
    wid = lax.axis_index("s") * _NC + lax.axis_index("c")
    base = wid * _B_PER_W

    pltpu.sync_copy(x_hbm.at[pl.ds(base, _B_PER_W)], idx_v.at[pl.ds(0, _B_PER_W)])
    for i in range(_B_PER_W // _L):
        sl = pl.ds(i * _L, _L)
        idx_v[sl] = lax.rem(idx_v[sl], _NUM_BUCKETS)

    def fire(q, slot):
        chunk = idx_v[pl.ds(q * _QS, _L)]
        for k in range(_QS):
            b = chunk[k]
            col = pl.multiple_of((b >> 7) << 7, 128)
            pltpu.async_copy(
                tableT_hbm.at[:, pl.ds(col, 128)], blocks.at[slot, k], sem
            )

    def wait_quad(slot):
        for k in range(_QS):
            pltpu.make_async_copy(
                tableT_hbm.at[:, pl.ds(0, 128)], blocks.at[slot, k], sem
            ).wait()

    def extract(q, slot):
        chunk = idx_v[pl.ds(q * _QS, _L)]
        for k in range(_QS):
            b = chunk[k]
            lanes = jnp.full((_L,), b & 127, jnp.int32)
            cols = jnp.full((_L,), q * _QS + k, jnp.int32)
            for r in range(_EMBED_DIM // _L):
                rows = lax.iota(jnp.int32, _L) + r * _L
                vals = plsc.load_gather(blocks.at[slot, k], [rows, lanes])
                plsc.store_scatter(stage, [rows, cols], vals)

    fire(0, 0)
    fire(1, 1)

    def step(q2, carry):
        q0 = q2 * 2
        wait_quad(0)
        extract(q0, 0)

        @pl.when(q0 + 2 < _NQ)
        def _():
            fire(q0 + 2, 0)

        wait_quad(1)
        extract(q0 + 1, 1)

        @pl.when(q0 + 3 < _NQ)
        def _():
            fire(q0 + 3, 1)

        return carry

    lax.fori_loop(0, _NQ // 2, step, 0)

    pltpu.sync_copy(stage, outT_hbm.at[:, pl.ds(base, _B_PER_W)])


@jax.jit
def kernel(x, table):
    fn = pl.kernel(
        _embed_body,
        out_type=jax.ShapeDtypeStruct((_EMBED_DIM, _BATCH), jnp.float32),
        mesh=plsc.VectorSubcoreMesh(core_axis_name="c", subcore_axis_name="s"),
        scratch_types=[
            pltpu.VMEM((_IDX_PAD,), jnp.int32),
            pltpu.VMEM((2, _QS, _EMBED_DIM, 128), jnp.float32),
            pltpu.VMEM((_EMBED_DIM, _B_PER_W), jnp.float32),
            pltpu.SemaphoreType.DMA,
        ],
        compiler_params=pltpu.CompilerParams(needs_layout_passes=False),
    )
    outT = fn(x, table.T)
    return outT.T
